# Initial kernel scaffold; baseline (speedup 1.0000x reference)
import jax
import jax.numpy as jnp
from jax.experimental import pallas as pl


def kernel(x, edge_index, W_l, W_r, b_l):
    raise NotImplementedError("write your pallas kernel here")


# speedup vs baseline: 4.0048x; 4.0048x over previous
import functools

import jax
import jax.numpy as jnp
from jax import lax
from jax.experimental import pallas as pl
from jax.experimental.pallas import tpu as pltpu
from jax.experimental.pallas import tpu_sc as plsc

NC = 2
NS = 16
L = 16
NW = NC * NS

CH = 128


def _sc_aggregate(N, E_pad, xst, src_cat, dst_p, zrows, z1):
    NA = N + 112
    SLC = NA // NS
    CPT = E_pad // CH // NS

    mesh = plsc.VectorSubcoreMesh(core_axis_name="c", subcore_axis_name="s")

    @functools.partial(
        pl.kernel,
        out_type=(
            jax.ShapeDtypeStruct((NC, NA, 128), jnp.float32),
            jax.ShapeDtypeStruct((NW * NA,), jnp.float32),
        ),
        mesh=mesh,
        scratch_types=[
            pltpu.VMEM((CH,), jnp.int32),
            pltpu.VMEM((CH,), jnp.int32),
            pltpu.VMEM((CH, 128), jnp.float32),
            pltpu.VMEM((NA,), jnp.float32),
            pltpu.SemaphoreType.DMA,
            pltpu.VMEM_SHARED((NA, 128), jnp.float32),
        ],
        compiler_params=pltpu.CompilerParams(needs_layout_passes=False),
    )
    def k(xst_h, src_h, dst_h, zrows_h, z1_h,
          agg_out, deg_out,
          src_v, dst_v, rows_v, deg_v, sem, acc_sh):
        c = lax.axis_index("c")
        s = lax.axis_index("s")
        w = s * NC + c

        rows_slice = pl.ds(s * SLC, SLC)
        pltpu.sync_copy(zrows_h, acc_sh.at[rows_slice])
        pltpu.sync_copy(z1_h, deg_v)
        plsc.subcore_barrier()

        one16 = jnp.ones((L,), jnp.float32)

        def body(j, carry):
            base = (s * CPT + j) * CH
            pltpu.sync_copy(src_h.at[pl.ds(c * E_pad + base, CH)], src_v)
            pltpu.sync_copy(dst_h.at[pl.ds(base, CH)], dst_v)
            gather = pltpu.async_copy(xst_h.at[src_v], rows_v, sem)
            for i in range(CH // L):
                ii = dst_v[pl.ds(i * L, L)]
                plsc.addupdate_scatter(deg_v, [ii], one16)
            gather.wait()
            pltpu.sync_copy(rows_v, acc_sh.at[dst_v], add=True)
            return carry

        lax.fori_loop(0, CPT, body, 0)
        plsc.subcore_barrier()

        pltpu.sync_copy(acc_sh.at[rows_slice], agg_out.at[c, rows_slice])
        pltpu.sync_copy(deg_v, deg_out.at[pl.ds(w * NA, NA)])

    return k(xst, src_cat, dst_p, zrows, z1)


def _tc_epilogue(agg, degm, x, W_l, W_r, b2):
    N, D = x.shape
    BN = 2048
    grid = (N + BN - 1) // BN

    def body(agg_ref, deg_ref, x_ref, wl_ref, wr_ref, b_ref, out_ref):
        a = jnp.concatenate([agg_ref[0], agg_ref[1]], axis=1)
        dsum = jnp.sum(deg_ref[...], axis=0, keepdims=True) * 0.5
        dinv = (1.0 / jnp.maximum(dsum, 1.0)).reshape(BN, 1)
        acc = lax.dot(a * dinv, wl_ref[...],
                      preferred_element_type=jnp.float32)
        acc = acc + lax.dot(x_ref[...], wr_ref[...],
                            preferred_element_type=jnp.float32)
        out_ref[...] = acc + b_ref[...]

    return pl.pallas_call(
        body,
        grid=(grid,),
        in_specs=[
            pl.BlockSpec((NC, BN, 128), lambda i: (0, i, 0)),
            pl.BlockSpec((NW, BN), lambda i: (0, i)),
            pl.BlockSpec((BN, D), lambda i: (i, 0)),
            pl.BlockSpec((D, D), lambda i: (0, 0)),
            pl.BlockSpec((D, D), lambda i: (0, 0)),
            pl.BlockSpec((1, D), lambda i: (0, 0)),
        ],
        out_specs=pl.BlockSpec((BN, D), lambda i: (i, 0)),
        out_shape=jax.ShapeDtypeStruct((N, D), jnp.float32),
    )(agg, degm, x, W_l, W_r, b2)


def kernel(x, edge_index, W_l, W_r, b_l):
    N, D = x.shape
    E = edge_index.shape[1]
    NP = N + 8
    NA = N + 112

    epc = NS * CH
    E_pad = ((E + epc - 1) // epc) * epc
    pad = E_pad - E
    src = jnp.concatenate(
        [edge_index[0], jnp.full((pad,), N, dtype=jnp.int32)])
    dst_p = jnp.concatenate(
        [edge_index[1], jnp.full((pad,), N, dtype=jnp.int32)])
    src_cat = jnp.concatenate([src, src + NP])

    xr = x.reshape(N, NC, 128).transpose(1, 0, 2)
    xr = jnp.pad(xr, ((0, 0), (0, NP - N), (0, 0)))
    xst = xr.reshape(NC * NP, 128)

    SLC = NA // NS
    zrows = jnp.zeros((SLC, 128), jnp.float32)
    z1 = jnp.zeros((NA,), jnp.float32)

    agg, deg_flat = _sc_aggregate(N, E_pad, xst, src_cat, dst_p, zrows, z1)
    degm = deg_flat.reshape(NW, NA)

    b2 = b_l.reshape(1, D)
    return _tc_epilogue(agg, degm, x, W_l, W_r, b2)
